# Initial kernel scaffold; baseline (speedup 1.0000x reference)
import jax
import jax.numpy as jnp
from jax.experimental import pallas as pl


def kernel(images, feat_0, W_conv, b_conv, W_cls, b_cls, W_bbox, b_bbox):
    raise NotImplementedError("write your pallas kernel here")


# speedup vs baseline: 12.9654x; 12.9654x over previous
import functools

import numpy as np
import jax
import jax.numpy as jnp
from jax.experimental import pallas as pl
from jax.experimental.pallas import tpu as pltpu

_SIZES = (32.0, 64.0, 128.0)
_ARS = (0.5, 1.0, 2.0)
_A = len(_SIZES) * len(_ARS)
_PRE_NMS = 2000
_POST_NMS = 1000
_NMS_THRESH = 0.7
_SCORE_THRESH = 0.0
_MIN_SIZE = 1e-3
_BBOX_XFORM_CLIP = float(np.log(1000.0 / 16.0))


def _head_kernel(x_ref, w_ref, bc_ref, wh_ref, bh_ref, out_ref):
    acc = jnp.zeros((4096, 256), jnp.float32)
    for dy in range(3):
        for dx in range(3):
            xp = x_ref[0, dy:dy + 64, dx:dx + 64, :].reshape(4096, 256)
            acc = acc + jnp.dot(xp, w_ref[dy * 3 + dx],
                                preferred_element_type=jnp.float32)
    t = jnp.maximum(acc + bc_ref[0][None, :], 0.0)
    out_ref[0] = jnp.dot(t, wh_ref[...],
                         preferred_element_type=jnp.float32) + bh_ref[0][None, :]


def _rpn_head(feat_0, W_conv, b_conv, W_cls, b_cls, W_bbox, b_bbox):
    B, C, H, W = feat_0.shape
    x = jnp.pad(feat_0.transpose(0, 2, 3, 1), ((0, 0), (1, 1), (1, 1), (0, 0)))
    wt = W_conv.transpose(2, 3, 1, 0).reshape(9, C, C)
    wh = jnp.concatenate([W_cls[:, :, 0, 0], W_bbox[:, :, 0, 0]], axis=0).T
    wh = jnp.pad(wh, ((0, 0), (0, 128 - wh.shape[1])))
    bh = jnp.pad(jnp.concatenate([b_cls, b_bbox]), (0, 128 - 5 * _A))

    out = pl.pallas_call(
        _head_kernel,
        grid=(B,),
        in_specs=[
            pl.BlockSpec((1, H + 2, W + 2, C), lambda b: (b, 0, 0, 0)),
            pl.BlockSpec((9, C, C), lambda b: (0, 0, 0)),
            pl.BlockSpec((1, C), lambda b: (0, 0)),
            pl.BlockSpec((C, 128), lambda b: (0, 0)),
            pl.BlockSpec((1, 128), lambda b: (0, 0)),
        ],
        out_specs=pl.BlockSpec((1, H * W, 128), lambda b: (b, 0, 0)),
        out_shape=jax.ShapeDtypeStruct((B, H * W, 128), jnp.float32),
    )(x, wt, b_conv[None, :], wh, bh[None, :])

    obj = out[:, :, :_A].reshape(B, H * W * _A)
    dl = out[:, :, _A:5 * _A].reshape(B, H * W * _A, 4)
    return obj, dl


def _make_anchors(H, W, stride):
    scales = jnp.array(_SIZES, jnp.float32)
    ar = jnp.array(_ARS, jnp.float32)
    h_ratios = jnp.sqrt(ar)
    w_ratios = 1.0 / h_ratios
    ws = (w_ratios[:, None] * scales[None, :]).reshape(-1)
    hs = (h_ratios[:, None] * scales[None, :]).reshape(-1)
    base = jnp.round(jnp.stack([-ws, -hs, ws, hs], axis=1) / 2.0)
    sx = jnp.arange(W, dtype=jnp.float32) * stride
    sy = jnp.arange(H, dtype=jnp.float32) * stride
    yy, xx = jnp.meshgrid(sy, sx, indexing="ij")
    shifts = jnp.stack([xx.reshape(-1), yy.reshape(-1), xx.reshape(-1), yy.reshape(-1)], axis=1)
    return (shifts[:, None, :] + base[None, :, :]).reshape(-1, 4)


def _decode(deltas, anchors):
    widths = anchors[:, 2] - anchors[:, 0]
    heights = anchors[:, 3] - anchors[:, 1]
    ctr_x = anchors[:, 0] + 0.5 * widths
    ctr_y = anchors[:, 1] + 0.5 * heights
    dx = deltas[..., 0]
    dy = deltas[..., 1]
    dw = jnp.minimum(deltas[..., 2], _BBOX_XFORM_CLIP)
    dh = jnp.minimum(deltas[..., 3], _BBOX_XFORM_CLIP)
    pcx = dx * widths + ctr_x
    pcy = dy * heights + ctr_y
    pw = jnp.exp(dw) * widths
    ph = jnp.exp(dh) * heights
    return jnp.stack([pcx - 0.5 * pw, pcy - 0.5 * ph, pcx + 0.5 * pw, pcy + 0.5 * ph], axis=-1)


_N = 2048
_M = 1024


def _nms_kernel(img_h, img_w, bt_ref, v_ref, out_ref, iou_ref):
    x1 = jnp.clip(bt_ref[0, 0, :], 0.0, img_w)
    y1 = jnp.clip(bt_ref[0, 1, :], 0.0, img_h)
    x2 = jnp.clip(bt_ref[0, 2, :], 0.0, img_w)
    y2 = jnp.clip(bt_ref[0, 3, :], 0.0, img_h)
    vals = v_ref[0, 0, :]
    scores = jax.nn.sigmoid(vals)
    wsz = x2 - x1
    hsz = y2 - y1
    valid = (wsz >= _MIN_SIZE) & (hsz >= _MIN_SIZE) & (scores >= _SCORE_THRESH)

    area = jnp.maximum(x2 - x1, 0.0) * jnp.maximum(y2 - y1, 0.0)
    ltx = jnp.maximum(x1[:, None], x1[None, :])
    lty = jnp.maximum(y1[:, None], y1[None, :])
    rbx = jnp.minimum(x2[:, None], x2[None, :])
    rby = jnp.minimum(y2[:, None], y2[None, :])
    whx = jnp.maximum(rbx - ltx, 0.0)
    why = jnp.maximum(rby - lty, 0.0)
    inter = whx * why
    iou_ref[...] = inter / jnp.maximum(area[:, None] + area[None, :] - inter, 1e-9)

    lanes = jax.lax.iota(jnp.int32, _N)
    keep0 = jnp.where(valid, 1.0, 0.0)

    def body(i, keep):
        row = iou_ref[pl.ds(i, 1), :].reshape(_N)
        keep_i = jnp.sum(jnp.where(lanes == i, keep, 0.0))
        sup = (row > _NMS_THRESH) & (lanes > i) & (keep_i > 0.0)
        return jnp.where(sup, 0.0, keep)

    keep = jax.lax.fori_loop(0, _PRE_NMS, body, keep0, unroll=False)

    validf = jnp.where(valid, 1.0, 0.0)
    supf = validf * (1.0 - keep)
    invf = 1.0 - validf
    nkeep = jnp.sum(keep)
    nsupp = jnp.sum(supf)
    tri = jnp.where(lanes[:, None] >= lanes[None, :], 1.0, 0.0)
    cols = jnp.concatenate(
        [keep[:, None], supf[:, None], invf[:, None], jnp.zeros((_N, 125), jnp.float32)],
        axis=1)
    cums = jnp.dot(tri, cols, preferred_element_type=jnp.float32)
    rank = jnp.where(
        keep > 0.0, cums[:, 0] - 1.0,
        jnp.where(valid, nkeep + cums[:, 1] - 1.0, nkeep + nsupp + cums[:, 2] - 1.0))
    outsc = jnp.where(keep > 0.0, scores, -1.0)
    payload = jnp.concatenate(
        [x1[:, None], y1[:, None], x2[:, None], y2[:, None], outsc[:, None],
         jnp.zeros((_N, 123), jnp.float32)], axis=1)
    rows = jax.lax.iota(jnp.int32, _M)
    onehot = jnp.where(rows[:, None].astype(jnp.float32) == rank[None, :], 1.0, 0.0)
    out_ref[0] = jnp.dot(onehot, payload, preferred_element_type=jnp.float32)


def _nms_stage(boxes_t, vals, img_h, img_w):
    B = boxes_t.shape[0]
    f = functools.partial(_nms_kernel, img_h, img_w)
    out = pl.pallas_call(
        f,
        grid=(B,),
        in_specs=[
            pl.BlockSpec((1, 4, _N), lambda b: (b, 0, 0)),
            pl.BlockSpec((1, 1, _N), lambda b: (b, 0, 0)),
        ],
        out_specs=pl.BlockSpec((1, _M, 128), lambda b: (b, 0, 0)),
        out_shape=jax.ShapeDtypeStruct((B, _M, 128), jnp.float32),
        scratch_shapes=[pltpu.VMEM((_N, _N), jnp.float32)],
    )(boxes_t, vals)
    return out[:, :_POST_NMS, :4], out[:, :_POST_NMS, 4]


def kernel(images, feat_0, W_conv, b_conv, W_cls, b_cls, W_bbox, b_bbox):
    B, _, img_h, img_w = images.shape
    _, C, H, W = feat_0.shape
    stride = float(img_h) / float(H)
    obj, dl = _rpn_head(feat_0, W_conv, b_conv, W_cls, b_cls, W_bbox, b_bbox)
    anchors = _make_anchors(H, W, stride)
    proposals = _decode(jax.lax.stop_gradient(dl), anchors)
    vals, idx = jax.lax.top_k(obj, _PRE_NMS)
    braw = jnp.take_along_axis(proposals, idx[..., None], axis=1)
    braw = jnp.pad(braw, ((0, 0), (0, _N - _PRE_NMS), (0, 0)))
    boxes_t = braw.transpose(0, 2, 1)
    vpad = jnp.pad(vals, ((0, 0), (0, _N - _PRE_NMS)), constant_values=-1e30)
    boxes, scores = _nms_stage(boxes_t, vpad[:, None, :], float(img_h), float(img_w))
    return boxes, scores
